# baseline (speedup 1.0000x reference)
import functools

import jax
import jax.numpy as jnp
from jax import lax
from jax.experimental import pallas as pl
from jax.experimental.pallas import tpu as pltpu
from jax.experimental.pallas import tpu_sc as plsc

GAMMA_EXP = 2
ROWS_PER_BLOCK = 1024
NSTREAM = 2
NSUB, LANES = 16, 16
CPAD = 1008


def _focal_rows(x, t, nclass):
    r = x.shape[0]
    m = jnp.max(x, axis=1, keepdims=True)
    s = jnp.sum(jnp.exp(x - m), axis=1, keepdims=True)
    cols = lax.broadcasted_iota(jnp.int32, (r, nclass), 1)
    pred_t = jnp.max(jnp.where(cols == t, x, -jnp.inf), axis=1, keepdims=True)
    logpt = pred_t - m - jnp.log(s)
    ce = -logpt
    pt = jnp.exp(logpt)
    return (1.0 - pt) ** GAMMA_EXP * ce


def _dense_body(*refs, nclass, nstream):
    ins = refs[:nstream]
    tins = refs[nstream:2 * nstream]
    outs = refs[2 * nstream:]
    for x_ref, t_ref, f_ref in zip(ins, tins, outs):
        fv = _focal_rows(x_ref[...], t_ref[...].astype(jnp.int32), nclass)
        f_ref[...] = fv.reshape(f_ref.shape)


def _sparse_body(tgt, fin, out, tgt_v, f_v, ones_v, z_v, red_v, acc_v,
                 cnt_sh, wsum_sh, *, chunks, bz):
    s = lax.axis_index("s")
    base = s * chunks

    pltpu.sync_copy(tgt.at[pl.ds(base, chunks)], tgt_v)
    pltpu.sync_copy(fin.at[pl.ds(base, chunks)], f_v)

    for j in range(chunks):
        for v in range(128 // LANES):
            sl = pl.ds(v * LANES, LANES)
            ones_v[j, sl] = jnp.full((LANES,), 1.0, jnp.float32)
    for v in range(CPAD // LANES):
        z_v[pl.ds(v * LANES, LANES)] = jnp.zeros((LANES,), jnp.float32)

    @pl.when(s == 0)
    def _zero():
        pltpu.sync_copy(z_v, cnt_sh)
        pltpu.sync_copy(z_v, wsum_sh)

    plsc.subcore_barrier()

    for j in range(chunks):
        pltpu.sync_copy(ones_v.at[j], cnt_sh.at[tgt_v.at[j]], add=True)
        pltpu.sync_copy(f_v.at[j], wsum_sh.at[tgt_v.at[j]], add=True)

    plsc.subcore_barrier()

    @pl.when(s == 0)
    def _final():
        pltpu.sync_copy(cnt_sh, red_v.at[0])
        pltpu.sync_copy(wsum_sh, red_v.at[1])

        inv_bz = 1.0 / bz

        def body(v, a):
            sl = pl.ds(v * LANES, LANES)
            return a + (1.0 - red_v[0, sl] * inv_bz) * red_v[1, sl]

        acc = lax.fori_loop(0, CPAD // LANES, body, jnp.zeros((LANES,), jnp.float32))
        total = jnp.sum(acc) * inv_bz
        acc_v[...] = jnp.full((LANES,), total, jnp.float32)
        pltpu.sync_copy(acc_v, out)


def kernel(pred, target):
    bz, nclass = pred.shape
    r = ROWS_PER_BLOCK
    ns = NSTREAM
    gsz = bz // r // ns
    chunks = bz // NSUB // 128
    fr = r // 128
    t2d = target.astype(jnp.int16).reshape(bz, 1)

    def mk_in(k):
        return pl.BlockSpec((r, nclass), lambda i, k=k: (i + k * gsz, 0))

    def mk_t(k):
        return pl.BlockSpec((r, 1), lambda i, k=k: (i + k * gsz, 0))

    f = pl.pallas_call(
        functools.partial(_dense_body, nclass=nclass, nstream=ns),
        grid=(gsz,),
        in_specs=[mk_in(k) for k in range(ns)] + [mk_t(k) for k in range(ns)],
        out_specs=[pl.BlockSpec((1, fr, 128), lambda i: (i, 0, 0))] * ns,
        out_shape=[jax.ShapeDtypeStruct((gsz, fr, 128), jnp.float32)] * ns,
    )(*([pred] * ns + [t2d] * ns))
    fcat = jnp.concatenate(f, axis=0).reshape(128, 128)

    mesh = plsc.VectorSubcoreMesh(core_axis_name="c", subcore_axis_name="s",
                                  num_cores=1, num_subcores=NSUB)
    out = pl.kernel(
        functools.partial(_sparse_body, chunks=chunks, bz=float(bz)),
        out_type=jax.ShapeDtypeStruct((LANES,), jnp.float32),
        mesh=mesh,
        scratch_types=(
            pltpu.VMEM((chunks, 128), jnp.int32),
            pltpu.VMEM((chunks, 128), jnp.float32),
            pltpu.VMEM((chunks, 128), jnp.float32),
            pltpu.VMEM((CPAD,), jnp.float32),
            pltpu.VMEM((2, CPAD), jnp.float32),
            pltpu.VMEM((LANES,), jnp.float32),
            pltpu.VMEM_SHARED((CPAD,), jnp.float32),
            pltpu.VMEM_SHARED((CPAD,), jnp.float32),
        ),
        compiler_params=pltpu.CompilerParams(needs_layout_passes=False),
    )(target.astype(jnp.int32).reshape(128, 128), fcat)
    return out[0].reshape(())


# speedup vs baseline: 1.0321x; 1.0321x over previous
import functools

import jax
import jax.numpy as jnp
from jax import lax
from jax.experimental import pallas as pl
from jax.experimental.pallas import tpu as pltpu
from jax.experimental.pallas import tpu_sc as plsc

GAMMA_EXP = 2
ROWS_PER_BLOCK = 1024
NSTREAM = 2
NSUB, LANES = 16, 16
CPAD = 1008


def _focal_rows(x, t, nclass):
    r = x.shape[0]
    m = jnp.max(x, axis=1, keepdims=True)
    s = jnp.sum(jnp.exp(x - m), axis=1, keepdims=True)
    cols = lax.broadcasted_iota(jnp.int32, (r, nclass), 1)
    pred_t = jnp.max(jnp.where(cols == t, x, -jnp.inf), axis=1, keepdims=True)
    logpt = pred_t - m - jnp.log(s)
    ce = -logpt
    pt = jnp.exp(logpt)
    return (1.0 - pt) ** GAMMA_EXP * ce


def _dense_body(*refs, nclass, nstream):
    ins = refs[:nstream]
    tins = refs[nstream:2 * nstream]
    outs = refs[2 * nstream:]
    for x_ref, t_ref, f_ref in zip(ins, tins, outs):
        fv = _focal_rows(x_ref[...], t_ref[...], nclass)
        f_ref[...] = fv.reshape(f_ref.shape)


def _sparse_body(tgt, fin, out, tgt_v, f_v, ones_v, z_v, red_v, acc_v,
                 cnt_sh, wsum_sh, *, chunks, bz):
    s = lax.axis_index("s")
    base = s * chunks

    pltpu.sync_copy(tgt.at[pl.ds(base, chunks)], tgt_v)
    pltpu.sync_copy(fin.at[pl.ds(base, chunks)], f_v)

    for j in range(chunks):
        for v in range(128 // LANES):
            sl = pl.ds(v * LANES, LANES)
            ones_v[j, sl] = jnp.full((LANES,), 1.0, jnp.float32)
    for v in range(CPAD // LANES):
        z_v[pl.ds(v * LANES, LANES)] = jnp.zeros((LANES,), jnp.float32)

    @pl.when(s == 0)
    def _zero():
        pltpu.sync_copy(z_v, cnt_sh)
        pltpu.sync_copy(z_v, wsum_sh)

    plsc.subcore_barrier()

    for j in range(chunks):
        pltpu.sync_copy(ones_v.at[j], cnt_sh.at[tgt_v.at[j]], add=True)
        pltpu.sync_copy(f_v.at[j], wsum_sh.at[tgt_v.at[j]], add=True)

    plsc.subcore_barrier()

    @pl.when(s == 0)
    def _final():
        pltpu.sync_copy(cnt_sh, red_v.at[0])
        pltpu.sync_copy(wsum_sh, red_v.at[1])

        inv_bz = 1.0 / bz

        def body(v, a):
            sl = pl.ds(v * LANES, LANES)
            return a + (1.0 - red_v[0, sl] * inv_bz) * red_v[1, sl]

        acc = lax.fori_loop(0, CPAD // LANES, body, jnp.zeros((LANES,), jnp.float32))
        total = jnp.sum(acc) * inv_bz
        acc_v[...] = jnp.full((LANES,), total, jnp.float32)
        pltpu.sync_copy(acc_v, out)


def kernel(pred, target):
    bz, nclass = pred.shape
    r = ROWS_PER_BLOCK
    ns = NSTREAM
    gsz = bz // r // ns
    chunks = bz // NSUB // 128
    fr = r // 128
    t2d = target.astype(jnp.int32).reshape(bz, 1)

    def mk_in(k):
        return pl.BlockSpec((r, nclass), lambda i, k=k: (i + k * gsz, 0))

    def mk_t(k):
        return pl.BlockSpec((r, 1), lambda i, k=k: (i + k * gsz, 0))

    f = pl.pallas_call(
        functools.partial(_dense_body, nclass=nclass, nstream=ns),
        grid=(gsz,),
        in_specs=[mk_in(k) for k in range(ns)] + [mk_t(k) for k in range(ns)],
        out_specs=[pl.BlockSpec((1, fr, 128), lambda i: (i, 0, 0))] * ns,
        out_shape=[jax.ShapeDtypeStruct((gsz, fr, 128), jnp.float32)] * ns,
    )(*([pred] * ns + [t2d] * ns))
    fcat = jnp.concatenate(f, axis=0).reshape(128, 128)

    mesh = plsc.VectorSubcoreMesh(core_axis_name="c", subcore_axis_name="s",
                                  num_cores=1, num_subcores=NSUB)
    out = pl.kernel(
        functools.partial(_sparse_body, chunks=chunks, bz=float(bz)),
        out_type=jax.ShapeDtypeStruct((LANES,), jnp.float32),
        mesh=mesh,
        scratch_types=(
            pltpu.VMEM((chunks, 128), jnp.int32),
            pltpu.VMEM((chunks, 128), jnp.float32),
            pltpu.VMEM((chunks, 128), jnp.float32),
            pltpu.VMEM((CPAD,), jnp.float32),
            pltpu.VMEM((2, CPAD), jnp.float32),
            pltpu.VMEM((LANES,), jnp.float32),
            pltpu.VMEM_SHARED((CPAD,), jnp.float32),
            pltpu.VMEM_SHARED((CPAD,), jnp.float32),
        ),
        compiler_params=pltpu.CompilerParams(needs_layout_passes=False),
    )(target.astype(jnp.int32).reshape(128, 128), fcat)
    return out[0].reshape(())


# speedup vs baseline: 1.0406x; 1.0082x over previous
import functools

import jax
import jax.numpy as jnp
from jax import lax
from jax.experimental import pallas as pl
from jax.experimental.pallas import tpu as pltpu
from jax.experimental.pallas import tpu_sc as plsc

GAMMA_EXP = 2
ROWS_PER_BLOCK = 1024
NSTREAM = 2
NSUB, LANES = 16, 16
CPAD = 1008


def _focal_rows(x, t, nclass):
    r = x.shape[0]
    m = jnp.max(x, axis=1, keepdims=True)
    s = jnp.sum(jnp.exp(x - m), axis=1, keepdims=True)
    cols = lax.broadcasted_iota(jnp.int32, (r, nclass), 1)
    pred_t = jnp.max(jnp.where(cols == t, x, -jnp.inf), axis=1, keepdims=True)
    logpt = pred_t - m - jnp.log(s)
    ce = -logpt
    pt = jnp.exp(logpt)
    return (1.0 - pt) ** GAMMA_EXP * ce


def _dense_body(*refs, nclass, nstream):
    ins = refs[:nstream]
    tins = refs[nstream:2 * nstream]
    outs = refs[2 * nstream:]
    for x_ref, t_ref, f_ref in zip(ins, tins, outs):
        fv = _focal_rows(x_ref[...], t_ref[...], nclass)
        f_ref[...] = fv.reshape(f_ref.shape)


def _sparse_body(tgt, fin, out, tgt_v, f_v, ones_v, z_v, red_v, acc_v,
                 cnt_sh, wsum_sh, *, chunks, bz):
    s = lax.axis_index("s")
    base = s * chunks

    pltpu.sync_copy(tgt.at[pl.ds(base, chunks)], tgt_v)
    pltpu.sync_copy(fin.at[pl.ds(base, chunks)], f_v)

    for j in range(chunks):
        for v in range(128 // LANES):
            sl = pl.ds(v * LANES, LANES)
            ones_v[j, sl] = jnp.full((LANES,), 1.0, jnp.float32)
    for v in range(CPAD // LANES):
        z_v[pl.ds(v * LANES, LANES)] = jnp.zeros((LANES,), jnp.float32)

    @pl.when(s == 0)
    def _zero():
        pltpu.sync_copy(z_v, cnt_sh)
        pltpu.sync_copy(z_v, wsum_sh)

    plsc.subcore_barrier()

    for j in range(chunks):
        pltpu.sync_copy(ones_v.at[j], cnt_sh.at[tgt_v.at[j]], add=True)
        pltpu.sync_copy(f_v.at[j], wsum_sh.at[tgt_v.at[j]], add=True)

    plsc.subcore_barrier()

    @pl.when(s == 0)
    def _final():
        pltpu.sync_copy(cnt_sh, red_v.at[0])
        pltpu.sync_copy(wsum_sh, red_v.at[1])

        inv_bz = 1.0 / bz

        def body(v, a):
            sl = pl.ds(v * LANES, LANES)
            return a + (1.0 - red_v[0, sl] * inv_bz) * red_v[1, sl]

        acc = lax.fori_loop(0, CPAD // LANES, body, jnp.zeros((LANES,), jnp.float32))
        total = jnp.sum(acc) * inv_bz
        acc_v[...] = jnp.full((LANES,), total, jnp.float32)
        pltpu.sync_copy(acc_v, out)


def kernel(pred, target):
    bz, nclass = pred.shape
    r = ROWS_PER_BLOCK
    ns = NSTREAM
    gsz = bz // r // ns
    chunks = bz // NSUB // 128
    fr = r // 128
    t2d = target.astype(jnp.int32).reshape(bz, 1)

    def mk_in(k):
        return pl.BlockSpec((r, nclass), lambda i, k=k: (i + k * gsz, 0))

    def mk_t(k):
        return pl.BlockSpec((r, 1), lambda i, k=k: (i + k * gsz, 0))

    f = pl.pallas_call(
        functools.partial(_dense_body, nclass=nclass, nstream=ns),
        grid=(gsz,),
        in_specs=[mk_in(k) for k in range(ns)] + [mk_t(k) for k in range(ns)],
        out_specs=[pl.BlockSpec((1, fr, 128), lambda i: (i, 0, 0))] * ns,
        out_shape=[jax.ShapeDtypeStruct((gsz, fr, 128), jnp.float32)] * ns,
        compiler_params=pltpu.CompilerParams(
            dimension_semantics=("arbitrary",),
            vmem_limit_bytes=100 * 1024 * 1024,
        ),
    )(*([pred] * ns + [t2d] * ns))
    fcat = jnp.concatenate(f, axis=0).reshape(128, 128)

    mesh = plsc.VectorSubcoreMesh(core_axis_name="c", subcore_axis_name="s",
                                  num_cores=1, num_subcores=NSUB)
    out = pl.kernel(
        functools.partial(_sparse_body, chunks=chunks, bz=float(bz)),
        out_type=jax.ShapeDtypeStruct((LANES,), jnp.float32),
        mesh=mesh,
        scratch_types=(
            pltpu.VMEM((chunks, 128), jnp.int32),
            pltpu.VMEM((chunks, 128), jnp.float32),
            pltpu.VMEM((chunks, 128), jnp.float32),
            pltpu.VMEM((CPAD,), jnp.float32),
            pltpu.VMEM((2, CPAD), jnp.float32),
            pltpu.VMEM((LANES,), jnp.float32),
            pltpu.VMEM_SHARED((CPAD,), jnp.float32),
            pltpu.VMEM_SHARED((CPAD,), jnp.float32),
        ),
        compiler_params=pltpu.CompilerParams(needs_layout_passes=False),
    )(target.astype(jnp.int32).reshape(128, 128), fcat)
    return out[0].reshape(())
